# baseline (speedup 1.0000x reference)
import jax
import jax.numpy as jnp
from jax.experimental import pallas as pl


def _tok_kernel(fn_ref, fr_ref, wn_ref, wr_ref, cn_ref, cr_ref, pos_ref, out_ref):
    d, n = pos_ref.shape
    tb = fn_ref.shape[0]
    base_n = pos_ref[...] + cn_ref[:, 0][:, None]
    base_r = pos_ref[...] + cr_ref[:, 0][:, None]
    f_n = jnp.broadcast_to(fn_ref[...][:, None, :], (tb, d, n))
    f_r = jnp.broadcast_to(fr_ref[...][:, None, :], (tb, d, n))
    out_ref[:, :, :n] = f_n * wn_ref[:, 0][None, :, None] + base_n[None, :, :]
    out_ref[:, :, n:] = f_r * wr_ref[:, 0][None, :, None] + base_r[None, :, :]


def kernel(features_nir, features_raman, W_nir, b_nir, W_raman, b_raman, pos_table, spec_table):
    B, N = features_nir.shape
    D = pos_table.shape[1]
    TB = 16

    pos_t = pos_table.T
    c_n = (b_nir + spec_table[0])[:, None]
    c_r = (b_raman + spec_table[1])[:, None]

    grid = (B // TB,)
    out_t = pl.pallas_call(
        _tok_kernel,
        grid=grid,
        in_specs=[
            pl.BlockSpec((TB, N), lambda i: (i, 0)),
            pl.BlockSpec((TB, N), lambda i: (i, 0)),
            pl.BlockSpec((D, 1), lambda i: (0, 0)),
            pl.BlockSpec((D, 1), lambda i: (0, 0)),
            pl.BlockSpec((D, 1), lambda i: (0, 0)),
            pl.BlockSpec((D, 1), lambda i: (0, 0)),
            pl.BlockSpec((D, N), lambda i: (0, 0)),
        ],
        out_specs=pl.BlockSpec((TB, D, 2 * N), lambda i: (i, 0, 0)),
        out_shape=jax.ShapeDtypeStruct((B, D, 2 * N), features_nir.dtype),
    )(features_nir, features_raman, W_nir, W_raman, c_n, c_r, pos_t)
    return out_t.transpose(0, 2, 1)


# speedup vs baseline: 1.0647x; 1.0647x over previous
import jax
import jax.numpy as jnp
from jax.experimental import pallas as pl


def _tok_kernel(fn_ref, fr_ref, wn_ref, wr_ref, cn_ref, cr_ref, pos_ref, out_ref):
    d, n = pos_ref.shape
    tb = fn_ref.shape[0]
    base_n = pos_ref[...] + cn_ref[:, 0][:, None]
    base_r = pos_ref[...] + cr_ref[:, 0][:, None]
    f_n = jnp.broadcast_to(fn_ref[...][:, None, :], (tb, d, n))
    f_r = jnp.broadcast_to(fr_ref[...][:, None, :], (tb, d, n))
    out_ref[:, :, :n] = f_n * wn_ref[:, 0][None, :, None] + base_n[None, :, :]
    out_ref[:, :, n:] = f_r * wr_ref[:, 0][None, :, None] + base_r[None, :, :]


def kernel(features_nir, features_raman, W_nir, b_nir, W_raman, b_raman, pos_table, spec_table):
    B, N = features_nir.shape
    D = pos_table.shape[1]
    TB = 64

    pos_t = pos_table.T
    c_n = (b_nir + spec_table[0])[:, None]
    c_r = (b_raman + spec_table[1])[:, None]

    grid = (B // TB,)
    out_t = pl.pallas_call(
        _tok_kernel,
        grid=grid,
        in_specs=[
            pl.BlockSpec((TB, N), lambda i: (i, 0)),
            pl.BlockSpec((TB, N), lambda i: (i, 0)),
            pl.BlockSpec((D, 1), lambda i: (0, 0)),
            pl.BlockSpec((D, 1), lambda i: (0, 0)),
            pl.BlockSpec((D, 1), lambda i: (0, 0)),
            pl.BlockSpec((D, 1), lambda i: (0, 0)),
            pl.BlockSpec((D, N), lambda i: (0, 0)),
        ],
        out_specs=pl.BlockSpec((TB, D, 2 * N), lambda i: (i, 0, 0)),
        out_shape=jax.ShapeDtypeStruct((B, D, 2 * N), features_nir.dtype),
    )(features_nir, features_raman, W_nir, W_raman, c_n, c_r, pos_t)
    return out_t.transpose(0, 2, 1)
